# baseline (speedup 1.0000x reference)
import functools

import jax
import jax.numpy as jnp
from jax import lax
from jax.experimental import pallas as pl
from jax.experimental.pallas import tpu as pltpu
from jax.experimental.pallas import tpu_sc as plsc

_NUM_CLASSES = 6
_DIM = 64
_BATCH = 16384
_NC = 2
_NS = 16
_L = 16
_NW = _NC * _NS
_BPW = _BATCH // _NW
_CHUNK = 128
_NCHUNK = _BPW // _CHUNK
_GROUPS = _BPW // _L

_mesh = plsc.VectorSubcoreMesh(
    core_axis_name="c", subcore_axis_name="s", num_cores=_NC, num_subcores=_NS
)


@functools.partial(
    pl.kernel,
    mesh=_mesh,
    out_type=jax.ShapeDtypeStruct((_BATCH, _NUM_CLASSES), jnp.float32),
    scratch_types=[
        pltpu.VMEM((_NCHUNK, _CHUNK), jnp.int32),
        pltpu.VMEM((_BPW, _DIM), jnp.float32),
        pltpu.VMEM((_NUM_CLASSES, _DIM * _L), jnp.float32),
        pltpu.VMEM((1, 128), jnp.float32),
        pltpu.VMEM((_BPW, _NUM_CLASSES), jnp.float32),
        pltpu.SemaphoreType.DMA,
    ],
    compiler_params=pltpu.CompilerParams(
        needs_layout_passes=False, use_tc_tiling_on_sc=False
    ),
)
def _emb_linear(idx_hbm, emb_hbm, wsp_hbm, bsp_hbm, out_hbm,
                idx_v, rows_v, w_v, b_v, out_v, gsem):
    wid = lax.axis_index("s") * _NC + lax.axis_index("c")

    pltpu.sync_copy(idx_hbm.at[pl.ds(wid * _NCHUNK, _NCHUNK)], idx_v)
    pltpu.sync_copy(wsp_hbm, w_v)
    pltpu.sync_copy(bsp_hbm, b_v)

    copies = []
    for j in range(_NCHUNK):
        copies.append(
            pltpu.async_copy(
                emb_hbm.at[idx_v.at[j]],
                rows_v.at[pl.ds(j * _CHUNK, _CHUNK)],
                gsem,
            )
        )
    for cp in copies:
        cp.wait()

    iota = lax.iota(jnp.int32, _L)

    def group_body(g, carry):
        rows = g * _L + iota
        accs = [b_v[0, pl.ds(c * _L, _L)] for c in range(_NUM_CLASSES)]
        for d in range(_DIM):
            col = jnp.full((_L,), d, jnp.int32)
            v = plsc.load_gather(rows_v, [rows, col])
            for c in range(_NUM_CLASSES):
                accs[c] = accs[c] + v * w_v[c, pl.ds(d * _L, _L)]
        for c in range(_NUM_CLASSES):
            plsc.store_scatter(out_v, [rows, jnp.full((_L,), c, jnp.int32)], accs[c])
        return carry

    lax.fori_loop(0, _GROUPS, group_body, 0)

    pltpu.sync_copy(out_v, out_hbm.at[pl.ds(wid * _BPW, _BPW), :])


def kernel(idx, emb, W, b):
    idx_r = idx.astype(jnp.int32).reshape(_NW * _NCHUNK, _CHUNK)
    wsp = jnp.broadcast_to(W.T[:, :, None], (_NUM_CLASSES, _DIM, _L))
    wsp = wsp.reshape(_NUM_CLASSES, _DIM * _L)
    bsp = jnp.broadcast_to(b[:, None], (_NUM_CLASSES, _L)).reshape(-1)
    bsp = jnp.pad(bsp, (0, 128 - _NUM_CLASSES * _L)).reshape(1, 128)
    return _emb_linear(idx_r, emb, wsp, bsp)


# speedup vs baseline: 1.5707x; 1.5707x over previous
import functools

import jax
import jax.numpy as jnp
from jax import lax
from jax.experimental import pallas as pl
from jax.experimental.pallas import tpu as pltpu
from jax.experimental.pallas import tpu_sc as plsc

_NUM_CLASSES = 6
_DIM = 64
_BATCH = 16384
_NC = 2
_NS = 16
_L = 16
_NW = _NC * _NS
_BPW = _BATCH // _NW
_CHUNK = 128
_NCHUNK = _BPW // _CHUNK
_WB = _NUM_CLASSES * _DIM * _L + 128

_mesh = plsc.VectorSubcoreMesh(
    core_axis_name="c", subcore_axis_name="s", num_cores=_NC, num_subcores=_NS
)


@functools.partial(
    pl.kernel,
    mesh=_mesh,
    out_type=jax.ShapeDtypeStruct((_BATCH, _NUM_CLASSES), jnp.float32),
    scratch_types=[
        pltpu.VMEM((_BPW + _L,), jnp.int32),
        pltpu.VMEM((_CHUNK, _DIM), jnp.float32),
        pltpu.VMEM((_WB,), jnp.float32),
        pltpu.VMEM((_BPW, _NUM_CLASSES), jnp.float32),
        pltpu.SemaphoreType.DMA,
    ],
    compiler_params=pltpu.CompilerParams(needs_layout_passes=False),
)
def _emb_linear(idx_hbm, emb_hbm, wb_hbm, out_hbm,
                idx_v, rows_v, wb_v, out_v, gsem):
    wid = lax.axis_index("s") * _NC + lax.axis_index("c")

    pltpu.sync_copy(idx_hbm.at[pl.ds(wid * _BPW, _BPW)], idx_v.at[pl.ds(0, _BPW)])
    pltpu.sync_copy(wb_hbm, wb_v)

    iota = lax.iota(jnp.int32, _L)

    def chunk_body(ck, carry):
        cbase = ck * _CHUNK

        def issue_body(t, carry2):
            tv = idx_v[pl.ds(cbase + t * 8, _L)]
            base = t * 8
            for l in range(8):
                pltpu.async_copy(emb_hbm.at[tv[l]], rows_v.at[base + l], gsem)
            pltpu.make_async_copy(
                emb_hbm.at[pl.ds(0, 8)], rows_v.at[pl.ds(base, 8)], gsem
            ).wait()
            return carry2

        lax.fori_loop(0, _CHUNK // 8, issue_body, 0)

        def group_body(g, carry2):
            rows = g * _L + iota
            accs = [
                wb_v[pl.ds(_NUM_CLASSES * _DIM * _L + c * _L, _L)]
                for c in range(_NUM_CLASSES)
            ]
            for d in range(_DIM):
                col = jnp.full((_L,), d, jnp.int32)
                v = plsc.load_gather(rows_v, [rows, col])
                for c in range(_NUM_CLASSES):
                    accs[c] = accs[c] + v * wb_v[pl.ds((c * _DIM + d) * _L, _L)]
            orow = cbase + g * _L + iota
            for c in range(_NUM_CLASSES):
                plsc.store_scatter(
                    out_v, [orow, jnp.full((_L,), c, jnp.int32)], accs[c]
                )
            return carry2

        lax.fori_loop(0, _CHUNK // _L, group_body, 0)
        return carry

    lax.fori_loop(0, _NCHUNK, chunk_body, 0)

    pltpu.sync_copy(out_v, out_hbm.at[pl.ds(wid * _BPW, _BPW), :])


def kernel(idx, emb, W, b):
    wsp = jnp.broadcast_to(W.T[:, :, None], (_NUM_CLASSES, _DIM, _L)).reshape(-1)
    bsp = jnp.broadcast_to(b[:, None], (_NUM_CLASSES, _L)).reshape(-1)
    bsp = jnp.pad(bsp, (0, 128 - _NUM_CLASSES * _L))
    wb = jnp.concatenate([wsp, bsp])
    return _emb_linear(idx.astype(jnp.int32), emb, wb)


# speedup vs baseline: 2.2412x; 1.4269x over previous
import functools

import jax
import jax.numpy as jnp
from jax import lax
from jax.experimental import pallas as pl
from jax.experimental.pallas import tpu as pltpu
from jax.experimental.pallas import tpu_sc as plsc

_NUM_CLASSES = 6
_DIM = 64
_BATCH = 16384
_NC = 2
_NS = 16
_L = 16
_NW = _NC * _NS
_BPW = _BATCH // _NW
_NRING = 2
_CPR = 128
_WB = _NUM_CLASSES * _DIM * _L + 128

_mesh = plsc.VectorSubcoreMesh(
    core_axis_name="c", subcore_axis_name="s", num_cores=_NC, num_subcores=_NS
)


_GDN = lax.GatherDimensionNumbers(
    offset_dims=(), collapsed_slice_dims=(0,), start_index_map=(0,)
)


def _take16(vec, idxvec):
    return lax.gather(
        vec, idxvec[:, None], _GDN, (1,),
        mode=lax.GatherScatterMode.PROMISE_IN_BOUNDS,
    )


def _lane(vec, pos):
    return _take16(vec, jnp.full((_L,), pos, jnp.int32))[0]


@functools.partial(
    pl.kernel,
    mesh=_mesh,
    out_type=jax.ShapeDtypeStruct((_BATCH, _NUM_CLASSES), jnp.float32),
    scratch_types=[
        pltpu.VMEM((_BPW + _L,), jnp.int32),
        pltpu.VMEM((_BPW + _L,), jnp.int32),
        pltpu.VMEM((_BPW + _L,), jnp.int32),
        pltpu.VMEM((_NRING, 8, 8, 128), jnp.float32),
        pltpu.VMEM((_CPR, _DIM), jnp.float32),
        pltpu.VMEM((_WB,), jnp.float32),
        pltpu.VMEM((_BPW, _NUM_CLASSES), jnp.float32),
        pltpu.SemaphoreType.DMA,
        pltpu.SemaphoreType.DMA,
    ],
    compiler_params=pltpu.CompilerParams(
        needs_layout_passes=False, internal_scratch_in_bytes=65536
    ),
)
def _emb_linear(sidx_hbm, perm_hbm, embt_hbm, wb_hbm, out_hbm,
                sidx_v, perm_v, blist_v, ring_v, rows_v, wb_v, out_v,
                gsem, osem):
    wid = lax.axis_index("s") * _NC + lax.axis_index("c")
    base = wid * _BPW

    pltpu.sync_copy(sidx_hbm.at[pl.ds(base, _BPW)], sidx_v.at[pl.ds(0, _BPW)])
    pltpu.sync_copy(perm_hbm.at[pl.ds(base, _BPW)], perm_v.at[pl.ds(0, _BPW)])
    pltpu.sync_copy(wb_hbm, wb_v)

    iota = lax.iota(jnp.int32, _L)
    rot = jnp.where(iota == 0, 15, iota - 1)

    def scan_body(t, carry):
        nb, last_b = carry
        bv = lax.shift_right_logical(sidx_v[pl.ds(t * _L, _L)], 7)
        prev = jnp.where(iota == 0, jnp.full((_L,), last_b, jnp.int32),
                         _take16(bv, rot))
        nm = bv != prev
        plsc.store_compressed(blist_v.at[pl.ds(nb, _L)], bv, mask=nm)
        cnt = plsc.all_reduce_population_count(nm)
        return nb + cnt[0], _lane(bv, 15)

    nblk, _ = lax.fori_loop(0, _BPW // _L, scan_body, (0, -1))

    def fire(j):
        bv = blist_v[pl.ds(lax.bitwise_and(j, ~(_L - 1)), _L)]
        bid = _lane(bv, lax.bitwise_and(j, _L - 1))
        slot = lax.bitwise_and(j, _NRING - 1)

        def tile_body(jt, carry):
            pltpu.async_copy(
                embt_hbm.at[jt, :, pl.ds(bid * 128, 128)],
                ring_v.at[slot, jt], gsem,
            )
            return carry

        lax.fori_loop(0, 8, tile_body, 0)

    def prime_body(j, carry):
        fire(j)
        return carry

    lax.fori_loop(0, jnp.minimum(_NRING - 1, nblk), prime_body, 0)

    jt_vecs = [lax.shift_right_logical(iota + 16 * q, 3) for q in range(4)]
    sub_vecs = [lax.bitwise_and(iota + 16 * q, 7) for q in range(4)]

    def chunk_body(ck, carry):
        def ext_body(kk, carry2):
            cur_bid, done = carry2
            k = ck * _CPR + kk
            ka = lax.bitwise_and(k, ~(_L - 1))
            kl = lax.bitwise_and(k, _L - 1)
            iv = sidx_v[pl.ds(ka, _L)]
            idx_k = _lane(iv, kl)
            bid_k = lax.shift_right_logical(idx_k, 7)
            new = bid_k != cur_bid

            @pl.when(new)
            def _():
                pltpu.make_async_copy(
                    embt_hbm.at[:, :, pl.ds(0, 128)],
                    ring_v.at[lax.bitwise_and(done, _NRING - 1)], gsem,
                ).wait()

                @pl.when(done + (_NRING - 1) < nblk)
                def _():
                    fire(done + (_NRING - 1))

            done = jnp.where(new, done + 1, done)
            slot = lax.bitwise_and(done - 1, _NRING - 1)
            lane = lax.bitwise_and(idx_k, 127)
            lane_splat = jnp.full((_L,), lane, jnp.int32)
            slot_splat = jnp.full((_L,), slot, jnp.int32)
            row_splat = jnp.full((_L,), kk, jnp.int32)
            for q in range(4):
                vq = plsc.load_gather(
                    ring_v, [slot_splat, jt_vecs[q], sub_vecs[q], lane_splat]
                )
                plsc.store_scatter(rows_v, [row_splat, iota + q * _L], vq)
            return bid_k, done

        carry = lax.fori_loop(0, _CPR, ext_body, carry)

        def group_body(g, carry2):
            rows = g * _L + iota
            accs = [
                wb_v[pl.ds(_NUM_CLASSES * _DIM * _L + c * _L, _L)]
                for c in range(_NUM_CLASSES)
            ]
            for d in range(_DIM):
                col = jnp.full((_L,), d, jnp.int32)
                v = plsc.load_gather(rows_v, [rows, col])
                for c in range(_NUM_CLASSES):
                    accs[c] = accs[c] + v * wb_v[pl.ds((c * _DIM + d) * _L, _L)]
            orow = ck * _CPR + g * _L + iota
            for c in range(_NUM_CLASSES):
                plsc.store_scatter(
                    out_v, [orow, jnp.full((_L,), c, jnp.int32)], accs[c]
                )
            return carry2

        lax.fori_loop(0, _CPR // _L, group_body, 0)
        return carry

    lax.fori_loop(0, _BPW // _CPR, chunk_body, (-1, 0))

    def out_body(k, carry):
        pv = perm_v[pl.ds(lax.bitwise_and(k, ~(_L - 1)), _L)]
        p = _lane(pv, lax.bitwise_and(k, _L - 1))
        pltpu.async_copy(out_v.at[k], out_hbm.at[p], osem)
        return carry

    lax.fori_loop(0, _BPW, out_body, 0)
    pltpu.make_async_copy(
        out_v, out_hbm.at[pl.ds(0, _BPW), :], osem
    ).wait()


def kernel(idx, emb, W, b):
    idx32 = idx.astype(jnp.int32)
    sidx, perm = lax.sort_key_val(idx32, lax.iota(jnp.int32, _BATCH))
    embt = emb.T.reshape(8, 8, 1000000)
    wsp = jnp.broadcast_to(W.T[:, :, None], (_NUM_CLASSES, _DIM, _L)).reshape(-1)
    bsp = jnp.broadcast_to(b[:, None], (_NUM_CLASSES, _L)).reshape(-1)
    bsp = jnp.pad(bsp, (0, 128 - _NUM_CLASSES * _L))
    wb = jnp.concatenate([wsp, bsp])
    return _emb_linear(sidx, perm, embt, wb)


# speedup vs baseline: 3.9786x; 1.7752x over previous
import functools

import jax
import jax.numpy as jnp
from jax import lax
from jax.experimental import pallas as pl
from jax.experimental.pallas import tpu as pltpu
from jax.experimental.pallas import tpu_sc as plsc

_NUM_CLASSES = 6
_DIM = 64
_BATCH = 16384
_NC = 2
_NS = 16
_L = 16
_NW = _NC * _NS
_BPW = _BATCH // _NW
_NRING = 4
_CPR = 128
_WB = _NUM_CLASSES * _DIM * _L + 128

_mesh = plsc.VectorSubcoreMesh(
    core_axis_name="c", subcore_axis_name="s", num_cores=_NC, num_subcores=_NS
)


_GDN = lax.GatherDimensionNumbers(
    offset_dims=(), collapsed_slice_dims=(0,), start_index_map=(0,)
)


def _take16(vec, idxvec):
    return lax.gather(
        vec, idxvec[:, None], _GDN, (1,),
        mode=lax.GatherScatterMode.PROMISE_IN_BOUNDS,
    )


def _lane(vec, pos):
    return _take16(vec, jnp.full((_L,), pos, jnp.int32))[0]


@functools.partial(
    pl.kernel,
    mesh=_mesh,
    out_type=jax.ShapeDtypeStruct((_BATCH, _NUM_CLASSES), jnp.float32),
    scratch_types=[
        pltpu.VMEM((_BPW + _L,), jnp.int32),
        pltpu.VMEM((_BPW + _L,), jnp.int32),
        pltpu.VMEM((_BPW + _L,), jnp.int32),
        pltpu.VMEM((_NRING, 8, 8, 128), jnp.float32),
        pltpu.VMEM((_CPR, _DIM), jnp.float32),
        pltpu.VMEM((_WB,), jnp.float32),
        pltpu.VMEM((_BPW, _NUM_CLASSES), jnp.float32),
        pltpu.SemaphoreType.DMA((_NRING,)),
        pltpu.SemaphoreType.DMA,
    ],
    compiler_params=pltpu.CompilerParams(
        needs_layout_passes=False, internal_scratch_in_bytes=65536
    ),
)
def _emb_linear(sidx_hbm, perm_hbm, embt_hbm, wb_hbm, out_hbm,
                sidx_v, perm_v, blist_v, ring_v, rows_v, wb_v, out_v,
                gsem, osem):
    wid = lax.axis_index("s") * _NC + lax.axis_index("c")
    base = wid * _BPW

    pltpu.sync_copy(sidx_hbm.at[pl.ds(base, _BPW)], sidx_v.at[pl.ds(0, _BPW)])
    pltpu.sync_copy(perm_hbm.at[pl.ds(base, _BPW)], perm_v.at[pl.ds(0, _BPW)])
    pltpu.sync_copy(wb_hbm, wb_v)

    iota = lax.iota(jnp.int32, _L)
    rot = jnp.where(iota == 0, 15, iota - 1)

    def scan_body(t, carry):
        nb, last_b = carry
        bv = lax.shift_right_logical(sidx_v[pl.ds(t * _L, _L)], 7)
        prev = jnp.where(iota == 0, jnp.full((_L,), last_b, jnp.int32),
                         _take16(bv, rot))
        nm = bv != prev
        plsc.store_compressed(blist_v.at[pl.ds(nb, _L)], bv, mask=nm)
        cnt = plsc.all_reduce_population_count(nm)
        return nb + cnt[0], _lane(bv, 15)

    nblk, _ = lax.fori_loop(0, _BPW // _L, scan_body, (0, -1))

    def fire(j):
        bv = blist_v[pl.ds(lax.bitwise_and(j, ~(_L - 1)), _L)]
        bid = _lane(bv, lax.bitwise_and(j, _L - 1))
        slot = lax.bitwise_and(j, _NRING - 1)

        def tile_body(jt, carry):
            pltpu.async_copy(
                embt_hbm.at[jt, :, pl.ds(bid * 128, 128)],
                ring_v.at[slot, jt], gsem.at[slot],
            )
            return carry

        lax.fori_loop(0, 8, tile_body, 0)

    def prime_body(j, carry):
        fire(j)
        return carry

    lax.fori_loop(0, jnp.minimum(_NRING - 1, nblk), prime_body, 0)

    jt_vecs = [lax.shift_right_logical(iota + 16 * q, 3) for q in range(4)]
    sub_vecs = [lax.bitwise_and(iota + 16 * q, 7) for q in range(4)]

    def chunk_body(ck, carry):
        def ext_body(kk, carry2):
            cur_bid, done = carry2
            k = ck * _CPR + kk
            ka = lax.bitwise_and(k, ~(_L - 1))
            kl = lax.bitwise_and(k, _L - 1)
            iv = sidx_v[pl.ds(ka, _L)]
            idx_k = _lane(iv, kl)
            bid_k = lax.shift_right_logical(idx_k, 7)
            new = bid_k != cur_bid

            @pl.when(new)
            def _():
                slot_d = lax.bitwise_and(done, _NRING - 1)
                pltpu.make_async_copy(
                    embt_hbm.at[:, :, pl.ds(0, 128)],
                    ring_v.at[slot_d], gsem.at[slot_d],
                ).wait()

                @pl.when(done + (_NRING - 1) < nblk)
                def _():
                    fire(done + (_NRING - 1))

            done = jnp.where(new, done + 1, done)
            slot = lax.bitwise_and(done - 1, _NRING - 1)
            lane = lax.bitwise_and(idx_k, 127)
            lane_splat = jnp.full((_L,), lane, jnp.int32)
            slot_splat = jnp.full((_L,), slot, jnp.int32)
            row_splat = jnp.full((_L,), kk, jnp.int32)
            for q in range(4):
                vq = plsc.load_gather(
                    ring_v, [slot_splat, jt_vecs[q], sub_vecs[q], lane_splat]
                )
                plsc.store_scatter(rows_v, [row_splat, iota + q * _L], vq)
            @pl.when(kl == _L - 1)
            def _():
                rows = (kk - (_L - 1)) + iota
                accs = [
                    wb_v[pl.ds(_NUM_CLASSES * _DIM * _L + c * _L, _L)]
                    for c in range(_NUM_CLASSES)
                ]
                for d in range(_DIM):
                    col = jnp.full((_L,), d, jnp.int32)
                    v = plsc.load_gather(rows_v, [rows, col])
                    for c in range(_NUM_CLASSES):
                        accs[c] = accs[c] + v * wb_v[pl.ds((c * _DIM + d) * _L, _L)]
                orow = ck * _CPR + rows
                for c in range(_NUM_CLASSES):
                    plsc.store_scatter(
                        out_v, [orow, jnp.full((_L,), c, jnp.int32)], accs[c]
                    )

            return bid_k, done

        carry = lax.fori_loop(0, _CPR, ext_body, carry)
        return carry

    lax.fori_loop(0, _BPW // _CPR, chunk_body, (-1, 0))

    def out_body(k, carry):
        pv = perm_v[pl.ds(lax.bitwise_and(k, ~(_L - 1)), _L)]
        p = _lane(pv, lax.bitwise_and(k, _L - 1))
        pltpu.async_copy(out_v.at[k], out_hbm.at[p], osem)
        return carry

    lax.fori_loop(0, _BPW, out_body, 0)
    pltpu.make_async_copy(
        out_v, out_hbm.at[pl.ds(0, _BPW), :], osem
    ).wait()


def kernel(idx, emb, W, b):
    idx32 = idx.astype(jnp.int32)
    sidx, perm = lax.sort_key_val(idx32, lax.iota(jnp.int32, _BATCH))
    embt = emb.T.reshape(8, 8, 1000000)
    wsp = jnp.broadcast_to(W.T[:, :, None], (_NUM_CLASSES, _DIM, _L)).reshape(-1)
    bsp = jnp.broadcast_to(b[:, None], (_NUM_CLASSES, _L)).reshape(-1)
    bsp = jnp.pad(bsp, (0, 128 - _NUM_CLASSES * _L))
    wb = jnp.concatenate([wsp, bsp])
    return _emb_linear(sidx, perm, embt, wb)


# speedup vs baseline: 4.2933x; 1.0791x over previous
import functools

import jax
import jax.numpy as jnp
from jax import lax
from jax.experimental import pallas as pl
from jax.experimental.pallas import tpu as pltpu
from jax.experimental.pallas import tpu_sc as plsc

_NUM_CLASSES = 6
_DIM = 64
_BATCH = 16384
_NC = 2
_NS = 16
_L = 16
_NW = _NC * _NS
_BPW = _BATCH // _NW
_NRING = 5
_CPR = 64
_WB = _NUM_CLASSES * _DIM * _L + 128

_mesh = plsc.VectorSubcoreMesh(
    core_axis_name="c", subcore_axis_name="s", num_cores=_NC, num_subcores=_NS
)


_GDN = lax.GatherDimensionNumbers(
    offset_dims=(), collapsed_slice_dims=(0,), start_index_map=(0,)
)


def _take16(vec, idxvec):
    return lax.gather(
        vec, idxvec[:, None], _GDN, (1,),
        mode=lax.GatherScatterMode.PROMISE_IN_BOUNDS,
    )


def _lane(vec, pos):
    return _take16(vec, jnp.full((_L,), pos, jnp.int32))[0]


@functools.partial(
    pl.kernel,
    mesh=_mesh,
    out_type=jax.ShapeDtypeStruct((_BATCH, _NUM_CLASSES), jnp.float32),
    scratch_types=[
        pltpu.VMEM((_BPW + _L,), jnp.int32),
        pltpu.VMEM((_BPW + _L,), jnp.int32),
        pltpu.VMEM((_BPW + _L,), jnp.int32),
        pltpu.VMEM((_NRING, 8, 8, 128), jnp.float32),
        pltpu.VMEM((_CPR, _DIM), jnp.float32),
        pltpu.VMEM((_WB,), jnp.float32),
        pltpu.VMEM((_BPW, _NUM_CLASSES), jnp.float32),
        pltpu.SemaphoreType.DMA((_NRING,)),
        pltpu.SemaphoreType.DMA,
    ],
    compiler_params=pltpu.CompilerParams(
        needs_layout_passes=False, internal_scratch_in_bytes=65536
    ),
)
def _emb_linear(sidx_hbm, perm_hbm, embt_hbm, wb_hbm, out_hbm,
                sidx_v, perm_v, blist_v, ring_v, rows_v, wb_v, out_v,
                gsem, osem):
    wid = lax.axis_index("s") * _NC + lax.axis_index("c")
    base = wid * _BPW

    pltpu.sync_copy(sidx_hbm.at[pl.ds(base, _BPW)], sidx_v.at[pl.ds(0, _BPW)])
    pltpu.sync_copy(perm_hbm.at[pl.ds(base, _BPW)], perm_v.at[pl.ds(0, _BPW)])
    pltpu.sync_copy(wb_hbm, wb_v)

    iota = lax.iota(jnp.int32, _L)
    rot = jnp.where(iota == 0, 15, iota - 1)

    def scan_body(t, carry):
        nb, last_b = carry
        bv = lax.shift_right_logical(sidx_v[pl.ds(t * _L, _L)], 7)
        prev = jnp.where(iota == 0, jnp.full((_L,), last_b, jnp.int32),
                         _take16(bv, rot))
        nm = bv != prev
        plsc.store_compressed(blist_v.at[pl.ds(nb, _L)], bv, mask=nm)
        cnt = plsc.all_reduce_population_count(nm)
        return nb + cnt[0], _lane(bv, 15)

    nblk, _ = lax.fori_loop(0, _BPW // _L, scan_body, (0, -1))

    def fire(j):
        bv = blist_v[pl.ds(lax.bitwise_and(j, ~(_L - 1)), _L)]
        bid = _lane(bv, lax.bitwise_and(j, _L - 1))
        slot = lax.rem(j, _NRING)

        def tile_body(jt, carry):
            pltpu.async_copy(
                embt_hbm.at[jt, :, pl.ds(bid * 128, 128)],
                ring_v.at[slot, jt], gsem.at[slot],
            )
            return carry

        lax.fori_loop(0, 8, tile_body, 0)

    def prime_body(j, carry):
        fire(j)
        return carry

    lax.fori_loop(0, jnp.minimum(_NRING - 1, nblk), prime_body, 0)

    jt_vecs = [lax.shift_right_logical(iota + 16 * q, 3) for q in range(4)]
    sub_vecs = [lax.bitwise_and(iota + 16 * q, 7) for q in range(4)]

    def chunk_body(ck, carry):
        def ext_body(kk, carry2):
            cur_bid, done = carry2
            k = ck * _CPR + kk
            ka = lax.bitwise_and(k, ~(_L - 1))
            kl = lax.bitwise_and(k, _L - 1)
            iv = sidx_v[pl.ds(ka, _L)]
            idx_k = _lane(iv, kl)
            bid_k = lax.shift_right_logical(idx_k, 7)
            new = bid_k != cur_bid

            @pl.when(new)
            def _():
                slot_d = lax.rem(done, _NRING)
                pltpu.make_async_copy(
                    embt_hbm.at[:, :, pl.ds(0, 128)],
                    ring_v.at[slot_d], gsem.at[slot_d],
                ).wait()

                @pl.when(done + (_NRING - 1) < nblk)
                def _():
                    fire(done + (_NRING - 1))

            done = jnp.where(new, done + 1, done)
            slot = lax.rem(done - 1, _NRING)
            lane = lax.bitwise_and(idx_k, 127)
            lane_splat = jnp.full((_L,), lane, jnp.int32)
            slot_splat = jnp.full((_L,), slot, jnp.int32)
            row_splat = jnp.full((_L,), kk, jnp.int32)
            for q in range(4):
                vq = plsc.load_gather(
                    ring_v, [slot_splat, jt_vecs[q], sub_vecs[q], lane_splat]
                )
                plsc.store_scatter(rows_v, [row_splat, iota + q * _L], vq)
            @pl.when(kl == _L - 1)
            def _():
                rows = (kk - (_L - 1)) + iota
                accs = [
                    wb_v[pl.ds(_NUM_CLASSES * _DIM * _L + c * _L, _L)]
                    for c in range(_NUM_CLASSES)
                ]
                for d in range(_DIM):
                    col = jnp.full((_L,), d, jnp.int32)
                    v = plsc.load_gather(rows_v, [rows, col])
                    for c in range(_NUM_CLASSES):
                        accs[c] = accs[c] + v * wb_v[pl.ds((c * _DIM + d) * _L, _L)]
                orow = ck * _CPR + rows
                for c in range(_NUM_CLASSES):
                    plsc.store_scatter(
                        out_v, [orow, jnp.full((_L,), c, jnp.int32)], accs[c]
                    )

            return bid_k, done

        carry = lax.fori_loop(0, _CPR, ext_body, carry)
        return carry

    lax.fori_loop(0, _BPW // _CPR, chunk_body, (-1, 0))

    def out_body(k, carry):
        pv = perm_v[pl.ds(lax.bitwise_and(k, ~(_L - 1)), _L)]
        p = _lane(pv, lax.bitwise_and(k, _L - 1))
        pltpu.async_copy(out_v.at[k], out_hbm.at[p], osem)
        return carry

    lax.fori_loop(0, _BPW, out_body, 0)
    pltpu.make_async_copy(
        out_v, out_hbm.at[pl.ds(0, _BPW), :], osem
    ).wait()


def kernel(idx, emb, W, b):
    idx32 = idx.astype(jnp.int32)
    sidx, perm = lax.sort_key_val(idx32, lax.iota(jnp.int32, _BATCH))
    embt = emb.T.reshape(8, 8, 1000000)
    wsp = jnp.broadcast_to(W.T[:, :, None], (_NUM_CLASSES, _DIM, _L)).reshape(-1)
    bsp = jnp.broadcast_to(b[:, None], (_NUM_CLASSES, _L)).reshape(-1)
    bsp = jnp.pad(bsp, (0, 128 - _NUM_CLASSES * _L))
    wb = jnp.concatenate([wsp, bsp])
    return _emb_linear(sidx, perm, embt, wb)


# speedup vs baseline: 4.5249x; 1.0539x over previous
import functools

import jax
import jax.numpy as jnp
from jax import lax
from jax.experimental import pallas as pl
from jax.experimental.pallas import tpu as pltpu
from jax.experimental.pallas import tpu_sc as plsc

_NUM_CLASSES = 6
_DIM = 64
_BATCH = 16384
_NC = 2
_NS = 16
_L = 16
_NW = _NC * _NS
_BPW = _BATCH // _NW
_NRING = 5
_CPR = 64
_WB = _NUM_CLASSES * _DIM * _L + 128

_mesh = plsc.VectorSubcoreMesh(
    core_axis_name="c", subcore_axis_name="s", num_cores=_NC, num_subcores=_NS
)


_GDN = lax.GatherDimensionNumbers(
    offset_dims=(), collapsed_slice_dims=(0,), start_index_map=(0,)
)


def _take16(vec, idxvec):
    return lax.gather(
        vec, idxvec[:, None], _GDN, (1,),
        mode=lax.GatherScatterMode.PROMISE_IN_BOUNDS,
    )


def _lane(vec, pos):
    return _take16(vec, jnp.full((_L,), pos, jnp.int32))[0]


@functools.partial(
    pl.kernel,
    mesh=_mesh,
    out_type=jax.ShapeDtypeStruct((_BATCH, _NUM_CLASSES), jnp.float32),
    scratch_types=[
        pltpu.VMEM((_BPW + _L,), jnp.int32),
        pltpu.VMEM((_BPW + _L,), jnp.int32),
        pltpu.VMEM((_BPW + _L,), jnp.int32),
        pltpu.VMEM((_NRING, 8, 8, 128), jnp.float32),
        pltpu.VMEM((_CPR, _DIM), jnp.float32),
        pltpu.VMEM((_WB,), jnp.float32),
        pltpu.VMEM((_BPW, _NUM_CLASSES), jnp.float32),
        pltpu.SemaphoreType.DMA((_NRING,)),
        pltpu.SemaphoreType.DMA,
    ],
    compiler_params=pltpu.CompilerParams(
        needs_layout_passes=False, internal_scratch_in_bytes=65536
    ),
)
def _emb_linear(sidx_hbm, perm_hbm, embt_hbm, wb_hbm, out_hbm,
                sidx_v, perm_v, blist_v, ring_v, rows_v, wb_v, out_v,
                gsem, osem):
    wid = lax.axis_index("s") * _NC + lax.axis_index("c")
    base = wid * _BPW

    pltpu.sync_copy(sidx_hbm.at[pl.ds(base, _BPW)], sidx_v.at[pl.ds(0, _BPW)])
    pltpu.sync_copy(perm_hbm.at[pl.ds(base, _BPW)], perm_v.at[pl.ds(0, _BPW)])
    pltpu.sync_copy(wb_hbm, wb_v)

    iota = lax.iota(jnp.int32, _L)
    rot = jnp.where(iota == 0, 15, iota - 1)

    def scan_body(t, carry):
        nb, last_b = carry
        bv = lax.shift_right_logical(sidx_v[pl.ds(t * _L, _L)], 7)
        prev = jnp.where(iota == 0, jnp.full((_L,), last_b, jnp.int32),
                         _take16(bv, rot))
        nm = bv != prev
        plsc.store_compressed(blist_v.at[pl.ds(nb, _L)], bv, mask=nm)
        cnt = plsc.all_reduce_population_count(nm)
        return nb + cnt[0], _lane(bv, 15)

    nblk, _ = lax.fori_loop(0, _BPW // _L, scan_body, (0, -1))

    def fire(j):
        bv = blist_v[pl.ds(lax.bitwise_and(j, ~(_L - 1)), _L)]
        bid = _lane(bv, lax.bitwise_and(j, _L - 1))
        slot = lax.rem(j, _NRING)

        def tile_body(jt, carry):
            pltpu.async_copy(
                embt_hbm.at[jt, :, pl.ds(bid * 128, 128)],
                ring_v.at[slot, jt], gsem.at[slot],
            )
            return carry

        lax.fori_loop(0, 8, tile_body, 0)

    def prime_body(j, carry):
        fire(j)
        return carry

    lax.fori_loop(0, jnp.minimum(_NRING - 1, nblk), prime_body, 0)

    jt_vecs = [lax.shift_right_logical(iota + 16 * q, 3) for q in range(4)]
    sub_vecs = [lax.bitwise_and(iota + 16 * q, 7) for q in range(4)]

    def chunk_body(ck, carry):
        def ext_body(kk, carry2):
            cur_bid, done = carry2
            k = ck * _CPR + kk
            ka = lax.bitwise_and(k, ~(_L - 1))
            kl = lax.bitwise_and(k, _L - 1)
            iv = sidx_v[pl.ds(ka, _L)]
            idx_k = _lane(iv, kl)
            bid_k = lax.shift_right_logical(idx_k, 7)
            new = bid_k != cur_bid

            @pl.when(new)
            def _():
                slot_d = lax.rem(done, _NRING)
                pltpu.make_async_copy(
                    embt_hbm.at[:, :, pl.ds(0, 128)],
                    ring_v.at[slot_d], gsem.at[slot_d],
                ).wait()

                @pl.when(done + (_NRING - 1) < nblk)
                def _():
                    fire(done + (_NRING - 1))

            done = jnp.where(new, done + 1, done)
            slot = lax.rem(done - 1, _NRING)
            lane = lax.bitwise_and(idx_k, 127)
            lane_splat = jnp.full((_L,), lane, jnp.int32)
            slot_splat = jnp.full((_L,), slot, jnp.int32)
            row_splat = jnp.full((_L,), kk, jnp.int32)
            for q in range(4):
                vq = plsc.load_gather(
                    ring_v, [slot_splat, jt_vecs[q], sub_vecs[q], lane_splat]
                )
                plsc.store_scatter(rows_v, [row_splat, iota + q * _L], vq)
            @pl.when(kl == _L - 1)
            def _():
                rows = (kk - (_L - 1)) + iota
                accs = [
                    wb_v[pl.ds(_NUM_CLASSES * _DIM * _L + c * _L, _L)]
                    for c in range(_NUM_CLASSES)
                ]
                for d in range(_DIM):
                    col = jnp.full((_L,), d, jnp.int32)
                    v = plsc.load_gather(rows_v, [rows, col])
                    for c in range(_NUM_CLASSES):
                        accs[c] = accs[c] + v * wb_v[pl.ds((c * _DIM + d) * _L, _L)]
                obase = ck * _CPR + kk - (_L - 1)
                orow = obase + iota
                for c in range(_NUM_CLASSES):
                    plsc.store_scatter(
                        out_v, [orow, jnp.full((_L,), c, jnp.int32)], accs[c]
                    )
                pvv = perm_v[pl.ds(obase, _L)]

                def w_body(l, c2):
                    p = _lane(pvv, l)
                    pltpu.async_copy(out_v.at[obase + l], out_hbm.at[p], osem)
                    return c2

                lax.fori_loop(0, _L, w_body, 0)

            return bid_k, done

        carry = lax.fori_loop(0, _CPR, ext_body, carry)
        return carry

    lax.fori_loop(0, _BPW // _CPR, chunk_body, (-1, 0))

    pltpu.make_async_copy(
        out_v, out_hbm.at[pl.ds(0, _BPW), :], osem
    ).wait()


def kernel(idx, emb, W, b):
    idx32 = idx.astype(jnp.int32)
    sidx, perm = lax.sort_key_val(idx32, lax.iota(jnp.int32, _BATCH))
    embt = emb.T.reshape(8, 8, 1000000)
    wsp = jnp.broadcast_to(W.T[:, :, None], (_NUM_CLASSES, _DIM, _L)).reshape(-1)
    bsp = jnp.broadcast_to(b[:, None], (_NUM_CLASSES, _L)).reshape(-1)
    bsp = jnp.pad(bsp, (0, 128 - _NUM_CLASSES * _L))
    wb = jnp.concatenate([wsp, bsp])
    return _emb_linear(sidx, perm, embt, wb)
